# baseline (speedup 1.0000x reference)
import jax
import jax.numpy as jnp
from jax.experimental import pallas as pl
from jax.experimental.pallas import tpu as pltpu

EMBED_DIM = 1024
NUM_EXPERTS = 64
ACTIVE_EXPERTS = 2
HIDDEN = 4 * EMBED_DIM
NTOK = 8
NPAIR = ACTIVE_EXPERTS * NTOK

BH = 1024
NH = HIDDEN // BH


def _router_body(x_ref, wr_ref, br_ref, eidx_ref, wts_ref):
    scores = jnp.maximum(
        jnp.dot(x_ref[:, 0, :], wr_ref[...], preferred_element_type=jnp.float32)
        + br_ref[...],
        0.0,
    )
    i0 = jnp.argmax(scores, axis=1)
    v0 = jnp.max(scores, axis=1)
    col = jax.lax.broadcasted_iota(jnp.int32, scores.shape, 1)
    masked = jnp.where(col == i0[:, None], -jnp.inf, scores)
    i1 = jnp.argmax(masked, axis=1)
    v1 = jnp.max(masked, axis=1)
    e1 = jnp.exp(v1 - v0)
    denom = 1.0 + e1
    w0 = 1.0 / denom
    w1 = e1 / denom

    eidx_ref[...] = jnp.concatenate([i0[:, None], i1[:, None]],
                                    axis=0).astype(jnp.int32)
    wts_ref[...] = jnp.concatenate([w0[:, None], w1[:, None]], axis=0)


def _expert_body(eidx_ref, wts_ref, x_ref, w1_ref, b1_ref, w2_ref, b2_ref,
                 out_ref):
    p = pl.program_id(0)
    h = pl.program_id(1)
    w = wts_ref[p, 0]

    hid = jnp.maximum(
        jnp.dot(x_ref[0], w1_ref[0], preferred_element_type=jnp.float32)
        + b1_ref[0],
        0.0,
    )
    part = jnp.dot(hid, w2_ref[0], preferred_element_type=jnp.float32)

    @pl.when(h == 0)
    def _first():
        out_ref[0] = w * (part + b2_ref[0])

    @pl.when(h != 0)
    def _acc():
        out_ref[0] += w * part


def _combine_body(part_ref, out_ref):
    out_ref[:, 0, :] = part_ref[0] + part_ref[1]


@jax.jit
def kernel(x, Wr, br, W1, b1, W2, b2):
    eidx, wts = pl.pallas_call(
        _router_body,
        out_shape=(
            jax.ShapeDtypeStruct((NPAIR, 1), jnp.int32),
            jax.ShapeDtypeStruct((NPAIR, 1), jnp.float32),
        ),
    )(x, Wr, br.reshape(1, NUM_EXPERTS))

    grid_spec = pltpu.PrefetchScalarGridSpec(
        num_scalar_prefetch=2,
        grid=(NPAIR, NH),
        in_specs=[
            pl.BlockSpec((1, 1, EMBED_DIM),
                         lambda p, h, eidx, wts: (p % NTOK, 0, 0)),
            pl.BlockSpec((1, EMBED_DIM, BH),
                         lambda p, h, eidx, wts: (eidx[p, 0], 0, h)),
            pl.BlockSpec((1, 1, BH),
                         lambda p, h, eidx, wts: (eidx[p, 0], 0, h)),
            pl.BlockSpec((1, BH, EMBED_DIM),
                         lambda p, h, eidx, wts: (eidx[p, 0], h, 0)),
            pl.BlockSpec((1, 1, EMBED_DIM),
                         lambda p, h, eidx, wts: (eidx[p, 0], 0, 0)),
        ],
        out_specs=pl.BlockSpec((1, 1, EMBED_DIM),
                               lambda p, h, eidx, wts: (p, 0, 0)),
    )

    partials = pl.pallas_call(
        _expert_body,
        grid_spec=grid_spec,
        out_shape=jax.ShapeDtypeStruct((NPAIR, 1, EMBED_DIM), jnp.float32),
        compiler_params=pltpu.CompilerParams(
            dimension_semantics=("parallel", "arbitrary")),
    )(eidx, wts, x, W1,
      b1.reshape(NUM_EXPERTS, 1, HIDDEN), W2,
      b2.reshape(NUM_EXPERTS, 1, EMBED_DIM))

    out = pl.pallas_call(
        _combine_body,
        out_shape=jax.ShapeDtypeStruct((NTOK, 1, EMBED_DIM), jnp.float32),
    )(partials.reshape(ACTIVE_EXPERTS, NTOK, EMBED_DIM))

    return out


# speedup vs baseline: 1.0639x; 1.0639x over previous
import jax
import jax.numpy as jnp
from jax.experimental import pallas as pl
from jax.experimental.pallas import tpu as pltpu

EMBED_DIM = 1024
NUM_EXPERTS = 64
ACTIVE_EXPERTS = 2
HIDDEN = 4 * EMBED_DIM
NTOK = 8
NPAIR = ACTIVE_EXPERTS * NTOK

BH = 1024
NH = HIDDEN // BH
NSTEP = NH * NPAIR


def _router_body(x_ref, wr_ref, br_ref, eidx_ref, tok_ref, wts_ref):
    scores = jnp.maximum(
        jnp.dot(x_ref[:, 0, :], wr_ref[...], preferred_element_type=jnp.float32)
        + br_ref[...],
        0.0,
    )
    i0 = jnp.argmax(scores, axis=1)
    v0 = jnp.max(scores, axis=1)
    col = jax.lax.broadcasted_iota(jnp.int32, scores.shape, 1)
    masked = jnp.where(col == i0[:, None], -jnp.inf, scores)
    i1 = jnp.argmax(masked, axis=1)
    v1 = jnp.max(masked, axis=1)
    e1 = jnp.exp(v1 - v0)
    denom = 1.0 + e1
    w0 = 1.0 / denom
    w1 = e1 / denom

    eidx_row = jnp.concatenate([i0[None, :], i1[None, :]], axis=1)
    wts_row = jnp.concatenate([w0[None, :], w1[None, :]], axis=1)
    pid_row = jax.lax.broadcasted_iota(jnp.int32, (1, NPAIR), 1)
    tok_row = pid_row % NTOK

    pid_col = jax.lax.broadcasted_iota(jnp.int32, (NPAIR, 1), 0)
    eidx_col = jnp.concatenate([i0[:, None], i1[:, None]], axis=0)
    key_col = eidx_col * NPAIR + pid_col
    key_row = eidx_row * NPAIR + pid_row
    lt = (key_col < key_row).astype(jnp.int32)
    rank_row = jnp.sum(lt, axis=0, keepdims=True)
    rr = jax.lax.broadcasted_iota(jnp.int32, (NPAIR, NPAIR), 0)
    P = (rr == rank_row).astype(jnp.int32)
    eidx_ref[...] = jnp.sum(P * eidx_row, axis=1, keepdims=True)
    tok_ref[...] = jnp.sum(P * tok_row, axis=1, keepdims=True)
    wts_ref[...] = jnp.sum(P.astype(jnp.float32) * wts_row, axis=1,
                           keepdims=True)


def _expert_body(eidx_ref, tok_ref, wts_ref, x_ref, b1_ref, b2_ref,
                 w1_hbm, w2_hbm, out_ref, w1buf, w2buf, sems):
    def _issue(s, slot):
        p = s % NPAIR
        h = s // NPAIR
        e = eidx_ref[p, 0]
        pltpu.make_async_copy(
            w1_hbm.at[e, :, pl.ds(h * BH, BH)], w1buf.at[slot],
            sems.at[0, slot]).start()
        pltpu.make_async_copy(
            w2_hbm.at[e, pl.ds(h * BH, BH), :], w2buf.at[slot],
            sems.at[1, slot]).start()

    def _wait(s, slot):
        p = s % NPAIR
        h = s // NPAIR
        e = eidx_ref[p, 0]
        pltpu.make_async_copy(
            w1_hbm.at[e, :, pl.ds(h * BH, BH)], w1buf.at[slot],
            sems.at[0, slot]).wait()
        pltpu.make_async_copy(
            w2_hbm.at[e, pl.ds(h * BH, BH), :], w2buf.at[slot],
            sems.at[1, slot]).wait()

    def _dup(s):
        p = s % NPAIR
        return jnp.logical_and(
            p != 0, eidx_ref[p, 0] == eidx_ref[jnp.maximum(p - 1, 0), 0])

    out_ref[...] = jnp.zeros_like(out_ref)
    _issue(0, 0)

    def _step(s, cur_slot):
        p = s % NPAIR
        h = s // NPAIR
        nxt = s + 1
        dup_next = _dup(nxt % NSTEP)
        issue_next = jnp.logical_and(nxt < NSTEP, jnp.logical_not(dup_next))

        @pl.when(issue_next)
        def _():
            _issue(nxt, 1 - cur_slot)

        @pl.when(jnp.logical_not(_dup(s)))
        def _():
            _wait(s, cur_slot)

        e = eidx_ref[p, 0]
        t = tok_ref[p, 0]
        w = wts_ref[p, 0]
        xrow = x_ref[pl.ds(t, 1), 0, :]
        hid = jnp.maximum(
            jnp.dot(xrow, w1buf[cur_slot], preferred_element_type=jnp.float32)
            + b1_ref[pl.ds(e, 1), pl.ds(h * BH, BH)],
            0.0,
        )
        part = jnp.dot(hid, w2buf[cur_slot],
                       preferred_element_type=jnp.float32)
        bias_gate = jnp.where(h == 0, w, 0.0)
        out_ref[pl.ds(t, 1), 0, :] += w * part + bias_gate * b2_ref[pl.ds(e, 1), :]
        return jnp.where(dup_next, cur_slot, 1 - cur_slot)

    jax.lax.fori_loop(0, NSTEP, _step, jnp.int32(0))


@jax.jit
def kernel(x, Wr, br, W1, b1, W2, b2):
    eidx, tok, wts = pl.pallas_call(
        _router_body,
        out_shape=(
            jax.ShapeDtypeStruct((NPAIR, 1), jnp.int32),
            jax.ShapeDtypeStruct((NPAIR, 1), jnp.int32),
            jax.ShapeDtypeStruct((NPAIR, 1), jnp.float32),
        ),
    )(x, Wr, br.reshape(1, NUM_EXPERTS))

    out = pl.pallas_call(
        _expert_body,
        in_specs=[
            pl.BlockSpec(memory_space=pltpu.SMEM),
            pl.BlockSpec(memory_space=pltpu.SMEM),
            pl.BlockSpec(memory_space=pltpu.SMEM),
            pl.BlockSpec(memory_space=pltpu.VMEM),
            pl.BlockSpec(memory_space=pltpu.VMEM),
            pl.BlockSpec(memory_space=pltpu.VMEM),
            pl.BlockSpec(memory_space=pltpu.HBM),
            pl.BlockSpec(memory_space=pltpu.HBM),
        ],
        out_specs=pl.BlockSpec(memory_space=pltpu.VMEM),
        out_shape=jax.ShapeDtypeStruct((NTOK, 1, EMBED_DIM), jnp.float32),
        scratch_shapes=[
            pltpu.VMEM((2, EMBED_DIM, BH), jnp.float32),
            pltpu.VMEM((2, BH, EMBED_DIM), jnp.float32),
            pltpu.SemaphoreType.DMA((2, 2)),
        ],
    )(eidx, tok, wts, x, b1, b2, W1, W2)

    return out


# speedup vs baseline: 1.1108x; 1.0441x over previous
import jax
import jax.numpy as jnp
from jax.experimental import pallas as pl
from jax.experimental.pallas import tpu as pltpu

EMBED_DIM = 1024
NUM_EXPERTS = 64
ACTIVE_EXPERTS = 2
HIDDEN = 4 * EMBED_DIM
NTOK = 8
NPAIR = ACTIVE_EXPERTS * NTOK

BH = 2048
NH = HIDDEN // BH
NSTEP = NH * NPAIR


def _router_body(x_ref, wr_ref, br_ref, eidx_ref, tok_ref, wts_ref):
    scores = jnp.maximum(
        jnp.dot(x_ref[:, 0, :], wr_ref[...], preferred_element_type=jnp.float32)
        + br_ref[...],
        0.0,
    )
    i0 = jnp.argmax(scores, axis=1)
    v0 = jnp.max(scores, axis=1)
    col = jax.lax.broadcasted_iota(jnp.int32, scores.shape, 1)
    masked = jnp.where(col == i0[:, None], -jnp.inf, scores)
    i1 = jnp.argmax(masked, axis=1)
    v1 = jnp.max(masked, axis=1)
    e1 = jnp.exp(v1 - v0)
    denom = 1.0 + e1
    w0 = 1.0 / denom
    w1 = e1 / denom

    eidx_row = jnp.concatenate([i0[None, :], i1[None, :]], axis=1)
    wts_row = jnp.concatenate([w0[None, :], w1[None, :]], axis=1)
    pid_row = jax.lax.broadcasted_iota(jnp.int32, (1, NPAIR), 1)
    tok_row = pid_row % NTOK

    pid_col = jax.lax.broadcasted_iota(jnp.int32, (NPAIR, 1), 0)
    eidx_col = jnp.concatenate([i0[:, None], i1[:, None]], axis=0)
    key_col = eidx_col * NPAIR + pid_col
    key_row = eidx_row * NPAIR + pid_row
    lt = (key_col < key_row).astype(jnp.int32)
    rank_row = jnp.sum(lt, axis=0, keepdims=True)
    rr = jax.lax.broadcasted_iota(jnp.int32, (NPAIR, NPAIR), 0)
    P = (rr == rank_row).astype(jnp.int32)
    eidx_ref[...] = jnp.sum(P * eidx_row, axis=1, keepdims=True)
    tok_ref[...] = jnp.sum(P * tok_row, axis=1, keepdims=True)
    wts_ref[...] = jnp.sum(P.astype(jnp.float32) * wts_row, axis=1,
                           keepdims=True)


def _expert_body(eidx_ref, tok_ref, wts_ref, x_ref, b1_ref, b2_ref,
                 w1_hbm, w2_hbm, out_ref, w1buf, w2buf, sems):
    def _issue(s, slot):
        p = s % NPAIR
        h = s // NPAIR
        e = eidx_ref[p, 0]
        pltpu.make_async_copy(
            w1_hbm.at[e, :, pl.ds(h * BH, BH)], w1buf.at[slot],
            sems.at[0, slot]).start()
        pltpu.make_async_copy(
            w2_hbm.at[e, pl.ds(h * BH, BH), :], w2buf.at[slot],
            sems.at[1, slot]).start()

    def _wait(s, slot):
        p = s % NPAIR
        h = s // NPAIR
        e = eidx_ref[p, 0]
        pltpu.make_async_copy(
            w1_hbm.at[e, :, pl.ds(h * BH, BH)], w1buf.at[slot],
            sems.at[0, slot]).wait()
        pltpu.make_async_copy(
            w2_hbm.at[e, pl.ds(h * BH, BH), :], w2buf.at[slot],
            sems.at[1, slot]).wait()

    def _dup(s):
        p = s % NPAIR
        return jnp.logical_and(
            p != 0, eidx_ref[p, 0] == eidx_ref[jnp.maximum(p - 1, 0), 0])

    out_ref[...] = jnp.zeros_like(out_ref)
    _issue(0, 0)

    def _step(s, cur_slot):
        p = s % NPAIR
        h = s // NPAIR
        nxt = s + 1
        dup_next = _dup(nxt % NSTEP)
        issue_next = jnp.logical_and(nxt < NSTEP, jnp.logical_not(dup_next))

        @pl.when(issue_next)
        def _():
            _issue(nxt, 1 - cur_slot)

        @pl.when(jnp.logical_not(_dup(s)))
        def _():
            _wait(s, cur_slot)

        e = eidx_ref[p, 0]
        t = tok_ref[p, 0]
        w = wts_ref[p, 0]
        xrow = x_ref[pl.ds(t, 1), 0, :]
        hid = jnp.maximum(
            jnp.dot(xrow, w1buf[cur_slot], preferred_element_type=jnp.float32)
            + b1_ref[pl.ds(e, 1), pl.ds(h * BH, BH)],
            0.0,
        )
        part = jnp.dot(hid, w2buf[cur_slot],
                       preferred_element_type=jnp.float32)
        bias_gate = jnp.where(h == 0, w, 0.0)
        out_ref[pl.ds(t, 1), 0, :] += w * part + bias_gate * b2_ref[pl.ds(e, 1), :]
        return jnp.where(dup_next, cur_slot, 1 - cur_slot)

    jax.lax.fori_loop(0, NSTEP, _step, jnp.int32(0))


@jax.jit
def kernel(x, Wr, br, W1, b1, W2, b2):
    eidx, tok, wts = pl.pallas_call(
        _router_body,
        out_shape=(
            jax.ShapeDtypeStruct((NPAIR, 1), jnp.int32),
            jax.ShapeDtypeStruct((NPAIR, 1), jnp.int32),
            jax.ShapeDtypeStruct((NPAIR, 1), jnp.float32),
        ),
    )(x, Wr, br.reshape(1, NUM_EXPERTS))

    out = pl.pallas_call(
        _expert_body,
        in_specs=[
            pl.BlockSpec(memory_space=pltpu.SMEM),
            pl.BlockSpec(memory_space=pltpu.SMEM),
            pl.BlockSpec(memory_space=pltpu.SMEM),
            pl.BlockSpec(memory_space=pltpu.VMEM),
            pl.BlockSpec(memory_space=pltpu.VMEM),
            pl.BlockSpec(memory_space=pltpu.VMEM),
            pl.BlockSpec(memory_space=pltpu.HBM),
            pl.BlockSpec(memory_space=pltpu.HBM),
        ],
        out_specs=pl.BlockSpec(memory_space=pltpu.VMEM),
        out_shape=jax.ShapeDtypeStruct((NTOK, 1, EMBED_DIM), jnp.float32),
        scratch_shapes=[
            pltpu.VMEM((2, EMBED_DIM, BH), jnp.float32),
            pltpu.VMEM((2, BH, EMBED_DIM), jnp.float32),
            pltpu.SemaphoreType.DMA((2, 2)),
        ],
    )(eidx, tok, wts, x, b1, b2, W1, W2)

    return out
